# baseline (speedup 1.0000x reference)
import functools

import numpy as np

import jax
import jax.numpy as jnp
from jax.experimental import pallas as pl
from jax.experimental.pallas import tpu as pltpu

_ROT0 = (13, 15, 26, 6)
_ROT1 = (17, 29, 16, 24)
_KS = (np.uint32(0), np.uint32(42),
       np.uint32(0) ^ np.uint32(42) ^ np.uint32(0x1BD11BDA))
_THRESHOLD = np.uint32(1258292)
_MASK_TOKEN = np.int32(2)

_COLS_PER_BLOCK = 256


def _threefry_bits(x1):
    x1 = x1 + _KS[1]
    x0 = x1
    x1 = ((x1 << np.uint32(13)) | (x1 >> np.uint32(19))) ^ x0
    for r in _ROT0[1:]:
        x0 = x0 + x1
        x1 = (x1 << np.uint32(r)) | (x1 >> np.uint32(32 - r))
        x1 = x1 ^ x0
    x0 = x0 + _KS[1]
    x1 = x1 + _KS[2] + np.uint32(1)
    for i in range(1, 5):
        for r in (_ROT0 if i % 2 == 0 else _ROT1):
            x0 = x0 + x1
            x1 = (x1 << np.uint32(r)) | (x1 >> np.uint32(32 - r))
            x1 = x1 ^ x0
        x0 = x0 + _KS[(i + 1) % 3]
        x1 = x1 + _KS[(i + 2) % 3] + np.uint32(i + 1)
    return x0 ^ x1


def _corrupt_block(seq_ref, out_ref, attn_out_ref, *, seq_len, cols):
    g = pl.program_id(0)
    base = (g * (cols * seq_len)).astype(jnp.uint32)
    s = jax.lax.broadcasted_iota(jnp.uint32, (seq_len, cols), 0)
    b = jax.lax.broadcasted_iota(jnp.uint32, (seq_len, cols), 1)
    flat = base + b * np.uint32(seq_len) + s
    bits = _threefry_bits(flat)
    corrupt = (bits >> np.uint32(9)) < _THRESHOLD
    out_ref[...] = jnp.where(corrupt, _MASK_TOKEN, seq_ref[...])
    attn_out_ref[...] = jnp.ones((seq_len, cols), jnp.float32)


def kernel(input_sequences, attention_mask):
    batch, seq_len = input_sequences.shape
    cols = _COLS_PER_BLOCK
    seq_t = input_sequences.T
    body = functools.partial(_corrupt_block, seq_len=seq_len, cols=cols)
    spec = pl.BlockSpec((seq_len, cols), lambda g: (0, g))
    corrupted_t, attn_out_t = pl.pallas_call(
        body,
        grid=(batch // cols,),
        in_specs=[spec],
        out_specs=[spec, spec],
        out_shape=[
            jax.ShapeDtypeStruct((seq_len, batch), jnp.int32),
            jax.ShapeDtypeStruct((seq_len, batch), jnp.float32),
        ],
        compiler_params=pltpu.CompilerParams(
            dimension_semantics=("arbitrary",)),
    )(seq_t)
    return corrupted_t.T, attn_out_t.T


# speedup vs baseline: 1.1759x; 1.1759x over previous
import functools

import numpy as np

import jax
import jax.numpy as jnp
from jax.experimental import pallas as pl
from jax.experimental.pallas import tpu as pltpu

_ROT0 = (13, 15, 26, 6)
_ROT1 = (17, 29, 16, 24)
_KS = (np.uint32(0), np.uint32(42),
       np.uint32(0) ^ np.uint32(42) ^ np.uint32(0x1BD11BDA))
_THRESHOLD = np.uint32(1258292)
_MASK_TOKEN = np.int32(2)

_COLS_PER_BLOCK = 2048


def _threefry_bits(x1):
    x1 = x1 + _KS[1]
    x0 = x1
    x1 = ((x1 << np.uint32(13)) | (x1 >> np.uint32(19))) ^ x0
    for r in _ROT0[1:]:
        x0 = x0 + x1
        x1 = (x1 << np.uint32(r)) | (x1 >> np.uint32(32 - r))
        x1 = x1 ^ x0
    x0 = x0 + _KS[1]
    x1 = x1 + _KS[2] + np.uint32(1)
    for i in range(1, 5):
        for r in (_ROT0 if i % 2 == 0 else _ROT1):
            x0 = x0 + x1
            x1 = (x1 << np.uint32(r)) | (x1 >> np.uint32(32 - r))
            x1 = x1 ^ x0
        x0 = x0 + _KS[(i + 1) % 3]
        x1 = x1 + _KS[(i + 2) % 3] + np.uint32(i + 1)
    return x0 ^ x1


def _corrupt_block(seq_ref, out_ref, attn_out_ref, *, seq_len, cols):
    g = pl.program_id(0)
    base = (g * (cols * seq_len)).astype(jnp.uint32)
    s = jax.lax.broadcasted_iota(jnp.uint32, (seq_len, cols), 0)
    b = jax.lax.broadcasted_iota(jnp.uint32, (seq_len, cols), 1)
    flat = base + b * np.uint32(seq_len) + s
    bits = _threefry_bits(flat)
    corrupt = (bits >> np.uint32(9)) < _THRESHOLD
    out_ref[...] = jnp.where(corrupt, _MASK_TOKEN, seq_ref[...])
    attn_out_ref[...] = jnp.ones((seq_len, cols), jnp.float32)


def kernel(input_sequences, attention_mask):
    batch, seq_len = input_sequences.shape
    cols = _COLS_PER_BLOCK
    seq_t = input_sequences.T
    body = functools.partial(_corrupt_block, seq_len=seq_len, cols=cols)
    spec = pl.BlockSpec((seq_len, cols), lambda g: (0, g))
    corrupted_t, attn_out_t = pl.pallas_call(
        body,
        grid=(batch // cols,),
        in_specs=[spec],
        out_specs=[spec, spec],
        out_shape=[
            jax.ShapeDtypeStruct((seq_len, batch), jnp.int32),
            jax.ShapeDtypeStruct((seq_len, batch), jnp.float32),
        ],
        compiler_params=pltpu.CompilerParams(
            dimension_semantics=("arbitrary",)),
    )(seq_t)
    return corrupted_t.T, attn_out_t.T


# speedup vs baseline: 1.2051x; 1.0248x over previous
import functools

import numpy as np

import jax
import jax.numpy as jnp
from jax.experimental import pallas as pl
from jax.experimental.pallas import tpu as pltpu

_ROT0 = (13, 15, 26, 6)
_ROT1 = (17, 29, 16, 24)
_KS = (np.uint32(0), np.uint32(42),
       np.uint32(0) ^ np.uint32(42) ^ np.uint32(0x1BD11BDA))
_THRESHOLD9 = np.uint32(1258292 << 9)
_MASK_TOKEN = np.int32(2)

_COLS_PER_BLOCK = 1024


def _threefry_bits(x1):
    x0 = x1
    x1 = ((x1 << np.uint32(13)) | (x1 >> np.uint32(19))) ^ x0
    for r in _ROT0[1:]:
        x0 = x0 + x1
        x1 = (x1 << np.uint32(r)) | (x1 >> np.uint32(32 - r))
        x1 = x1 ^ x0
    x0 = x0 + _KS[1]
    x1 = x1 + _KS[2] + np.uint32(1)
    for i in range(1, 5):
        for r in (_ROT0 if i % 2 == 0 else _ROT1):
            x0 = x0 + x1
            x1 = (x1 << np.uint32(r)) | (x1 >> np.uint32(32 - r))
            x1 = x1 ^ x0
        x0 = x0 + _KS[(i + 1) % 3]
        x1 = x1 + _KS[(i + 2) % 3] + np.uint32(i + 1)
    return x0 ^ x1


def _corrupt_block(seq_ref, out_ref, attn_out_ref, *, seq_len, cols):
    g = pl.program_id(0)
    base42 = (g * (cols * seq_len) + 42).astype(jnp.uint32)
    s = jax.lax.broadcasted_iota(jnp.uint32, (seq_len, cols), 0)
    b = jax.lax.broadcasted_iota(jnp.uint32, (seq_len, cols), 1)
    x1 = base42 + b * np.uint32(seq_len) + s
    bits = _threefry_bits(x1)
    corrupt = bits < _THRESHOLD9
    out_ref[...] = jnp.where(corrupt, _MASK_TOKEN, seq_ref[...])
    attn_out_ref[...] = jnp.ones((seq_len, cols), jnp.float32)


def kernel(input_sequences, attention_mask):
    batch, seq_len = input_sequences.shape
    cols = _COLS_PER_BLOCK
    seq_t = input_sequences.T
    body = functools.partial(_corrupt_block, seq_len=seq_len, cols=cols)
    spec = pl.BlockSpec((seq_len, cols), lambda g: (0, g))
    corrupted_t, attn_out_t = pl.pallas_call(
        body,
        grid=(batch // cols,),
        in_specs=[spec],
        out_specs=[spec, spec],
        out_shape=[
            jax.ShapeDtypeStruct((seq_len, batch), jnp.int32),
            jax.ShapeDtypeStruct((seq_len, batch), jnp.float32),
        ],
        compiler_params=pltpu.CompilerParams(
            dimension_semantics=("arbitrary",)),
    )(seq_t)
    return corrupted_t.T, attn_out_t.T


# speedup vs baseline: 1.2093x; 1.0035x over previous
import functools

import numpy as np

import jax
import jax.numpy as jnp
from jax.experimental import pallas as pl
from jax.experimental.pallas import tpu as pltpu

_ROT0 = (13, 15, 26, 6)
_ROT1 = (17, 29, 16, 24)
_KS = (np.uint32(0), np.uint32(42),
       np.uint32(0) ^ np.uint32(42) ^ np.uint32(0x1BD11BDA))
_THRESHOLD9 = np.uint32(1258292 << 9)
_MASK_TOKEN = np.int32(2)

_COLS_PER_BLOCK = 1024


def _threefry_bits(x1):
    x0 = x1
    x1 = ((x1 << np.uint32(13)) | (x1 >> np.uint32(19))) ^ x0
    for r in _ROT0[1:]:
        x0 = x0 + x1
        x1 = (x1 << np.uint32(r)) | (x1 >> np.uint32(32 - r))
        x1 = x1 ^ x0
    x0 = x0 + _KS[1]
    x1 = x1 + _KS[2] + np.uint32(1)
    for i in range(1, 5):
        for r in (_ROT0 if i % 2 == 0 else _ROT1):
            x0 = x0 + x1
            x1 = (x1 << np.uint32(r)) | (x1 >> np.uint32(32 - r))
            x1 = x1 ^ x0
        x0 = x0 + _KS[(i + 1) % 3]
        x1 = x1 + _KS[(i + 2) % 3] + np.uint32(i + 1)
    return x0 ^ x1


def _corrupt_block(seq_ref, out_ref, attn_out_ref, rel_ref, *, seq_len, cols):
    g = pl.program_id(0)

    @pl.when(g == 0)
    def _init():
        s = jax.lax.broadcasted_iota(jnp.uint32, (seq_len, cols), 0)
        b = jax.lax.broadcasted_iota(jnp.uint32, (seq_len, cols), 1)
        rel_ref[...] = b * np.uint32(seq_len) + s

    base42 = (g * (cols * seq_len) + 42).astype(jnp.uint32)
    x1 = base42 + rel_ref[...]
    bits = _threefry_bits(x1)
    corrupt = bits < _THRESHOLD9
    out_ref[...] = jnp.where(corrupt, _MASK_TOKEN, seq_ref[...])
    attn_out_ref[...] = jnp.ones((seq_len, cols), jnp.float32)


def kernel(input_sequences, attention_mask):
    batch, seq_len = input_sequences.shape
    cols = _COLS_PER_BLOCK
    seq_t = input_sequences.T
    body = functools.partial(_corrupt_block, seq_len=seq_len, cols=cols)
    spec = pl.BlockSpec((seq_len, cols), lambda g: (0, g))
    corrupted_t, attn_out_t = pl.pallas_call(
        body,
        grid=(batch // cols,),
        in_specs=[spec],
        out_specs=[spec, spec],
        out_shape=[
            jax.ShapeDtypeStruct((seq_len, batch), jnp.int32),
            jax.ShapeDtypeStruct((seq_len, batch), jnp.float32),
        ],
        scratch_shapes=[pltpu.VMEM((seq_len, cols), jnp.uint32)],
        compiler_params=pltpu.CompilerParams(
            dimension_semantics=("arbitrary",)),
    )(seq_t)
    return corrupted_t.T, attn_out_t.T
